# baseline (speedup 1.0000x reference)
import functools

import jax
import jax.numpy as jnp
from jax import lax
from jax.experimental import pallas as pl
from jax.experimental.pallas import tpu as pltpu
from jax.experimental.pallas import tpu_sc as plsc

_LANES = 16
_TL = 128
_TS = 8


def _weights_body(p_ref, m_ref, w_ref, tpe_ref):
    m = m_ref[...]
    w = jnp.sum(p_ref[...] * m, axis=1, keepdims=True)
    w_ref[...] = jnp.broadcast_to(w, w_ref.shape)
    tpe_ref[...] = jnp.sum(m, axis=0, keepdims=True)


def _make_sc_scale(T, H, NC, NS):
    NW = NC * NS
    RW = T // NW
    CH = 64
    NCH = RW // CH
    HT = H // _TL
    mesh = plsc.VectorSubcoreMesh(core_axis_name="c", subcore_axis_name="s")

    @functools.partial(
        pl.kernel,
        mesh=mesh,
        out_type=jax.ShapeDtypeStruct((T, H), jnp.float32),
        scratch_types=[
            pltpu.VMEM((CH, H), jnp.float32),
            pltpu.VMEM((CH, _TL), jnp.float32),
        ],
        compiler_params=pltpu.CompilerParams(use_tc_tiling_on_sc=True),
    )
    def _sc_scale(hs_hbm, wexp_hbm, out_hbm, buf, wv):
        c = lax.axis_index("c")
        s = lax.axis_index("s")
        wid = s * NC + c
        base = wid * RW
        for k in range(NCH):
            row0 = base + k * CH
            pltpu.sync_copy(hs_hbm.at[pl.ds(row0, CH)], buf)
            pltpu.sync_copy(wexp_hbm.at[pl.ds(row0, CH)], wv)

            def row_body(r, carry):
                w16 = wv[r, pl.ds(0, _LANES)]
                for j in range(H // _LANES):
                    sl = pl.ds(j * _LANES, _LANES)
                    buf[r, sl] = buf[r, sl] * w16
                return carry

            lax.fori_loop(0, CH, row_body, 0)
            pltpu.sync_copy(buf, out_hbm.at[pl.ds(row0, CH)])

    return _sc_scale


def kernel(hidden_states, probs, routing_map):
    hidden_shape = hidden_states.shape
    H = hidden_shape[-1]
    T, E = probs.shape
    hs = hidden_states.reshape(T, H)
    mask = routing_map.astype(jnp.float32)

    wexp, tpe = pl.pallas_call(
        _weights_body,
        out_shape=[
            jax.ShapeDtypeStruct((T, _TL), jnp.float32),
            jax.ShapeDtypeStruct((1, E), jnp.float32),
        ],
    )(probs, mask)

    info = plsc.get_sparse_core_info()
    out = _make_sc_scale(T, H, info.num_cores, info.num_subcores)(hs, wexp)

    tokens_per_expert = tpe.reshape(E).astype(jnp.int32)
    return out.reshape(hidden_shape), tokens_per_expert


# speedup vs baseline: 7.3223x; 7.3223x over previous
import jax
import jax.numpy as jnp
from jax.experimental import pallas as pl
from jax.experimental.pallas import tpu as pltpu

_BS = 512


def _body(hs_ref, p_ref, m_ref, out_ref, tpe_ref):
    m = m_ref[...]
    w = jnp.sum(p_ref[...] * m, axis=1)
    bs, b, _ = hs_ref.shape
    out_ref[...] = hs_ref[...] * w.reshape(bs, b, 1)

    @pl.when(pl.program_id(0) == 0)
    def _init():
        tpe_ref[...] = jnp.zeros_like(tpe_ref)

    tpe_ref[...] += jnp.sum(m, axis=0, keepdims=True)


def kernel(hidden_states, probs, routing_map):
    S, B, H = hidden_states.shape
    T, E = probs.shape
    mask = routing_map.astype(jnp.float32)

    grid = (S // _BS,)
    out, tpe = pl.pallas_call(
        _body,
        grid=grid,
        in_specs=[
            pl.BlockSpec((_BS, B, H), lambda i: (i, 0, 0)),
            pl.BlockSpec((_BS * B, E), lambda i: (i, 0)),
            pl.BlockSpec((_BS * B, E), lambda i: (i, 0)),
        ],
        out_specs=[
            pl.BlockSpec((_BS, B, H), lambda i: (i, 0, 0)),
            pl.BlockSpec((1, E), lambda i: (0, 0)),
        ],
        out_shape=[
            jax.ShapeDtypeStruct((S, B, H), hidden_states.dtype),
            jax.ShapeDtypeStruct((1, E), jnp.float32),
        ],
    )(hidden_states, probs, mask)

    tokens_per_expert = tpe.reshape(E).astype(jnp.int32)
    return out, tokens_per_expert
